# baseline (speedup 1.0000x reference)
import functools

import jax
import jax.numpy as jnp
from jax import lax
from jax.experimental import pallas as pl
from jax.experimental.pallas import tpu as pltpu
from jax.experimental.pallas import tpu_sc as plsc

N_ATOMS = 10000
N_BONDS = 320000
MAX_NB = 32
ATOM_FDIM = 128
BOND_FDIM = 144
HIDDEN = 128
N_MOLS = 512
DEPTH = 3

NW = 32
ATOMS_PAD = 10240
ATOMS_PER_W = ATOMS_PAD // NW
GROUP_ATOMS = 2
GROUPS_SUM = ATOMS_PER_W // GROUP_ATOMS
GSUM_CHUNK = GROUP_ATOMS * MAX_NB
BONDS_PER_W = N_BONDS // NW
GCHUNK = 128
FULL_GROUPS = BONDS_PER_W // GCHUNK
LAST_OFF = BONDS_PER_W - GCHUNK


def _sc_mesh():
    return plsc.VectorSubcoreMesh(core_axis_name="c", subcore_axis_name="s")


IDX_PER_W = ATOMS_PER_W * MAX_NB


RCHUNK = 64
NSLOT_R = 4
PREF_R = 2
NG_R = IDX_PER_W // (2 * RCHUNK)


HALF_W = IDX_PER_W // 2


def _gather_raw_body(t1_hbm, t2_hbm, a2b_hbm, out_hbm, i1_all, i2_all,
                     b1_0, b1_1, b1_2, b1_3, b2_0, b2_1, b2_2, b2_3,
                     gs0, gs1, gs2, gs3, ws0, ws1, ws2, ws3):
    wid = lax.axis_index("s") * 2 + lax.axis_index("c")
    rbase = wid * IDX_PER_W
    pltpu.sync_copy(a2b_hbm.at[pl.ds(rbase, HALF_W)], i1_all)
    pltpu.sync_copy(a2b_hbm.at[pl.ds(rbase + HALF_W, HALF_W)], i2_all)

    b1s = (b1_0, b1_1, b1_2, b1_3)
    b2s = (b2_0, b2_1, b2_2, b2_3)
    gss = (gs0, gs1, gs2, gs3)
    wss = (ws0, ws1, ws2, ws3)

    def gath(g, b):
        o = g * RCHUNK
        return (pltpu.make_async_copy(
                    t1_hbm.at[i1_all.at[pl.ds(o, RCHUNK)]], b1s[b], gss[b]),
                pltpu.make_async_copy(
                    t2_hbm.at[i2_all.at[pl.ds(o, RCHUNK)]], b2s[b], gss[b]))

    def wr(g, b):
        o = g * RCHUNK
        return (pltpu.make_async_copy(
                    b1s[b], out_hbm.at[pl.ds(rbase + o, RCHUNK)], wss[b]),
                pltpu.make_async_copy(
                    b2s[b], out_hbm.at[pl.ds(rbase + HALF_W + o, RCHUNK)],
                    wss[b]))

    def start2(pair):
        pair[0].start()
        pair[1].start()

    def wait2(pair):
        pair[0].wait()
        pair[1].wait()

    for b in range(PREF_R):
        start2(gath(b, b))

    def step(g, b):
        wait2(gath(g, b))
        start2(wr(g, b))
        f = g + PREF_R
        fb = (b + PREF_R) % NSLOT_R

        @pl.when(f >= NSLOT_R)
        def _():
            wait2(wr(f - NSLOT_R, fb))

        @pl.when(f < NG_R)
        def _():
            start2(gath(f, fb))

    def outer(i, _):
        for b in range(NSLOT_R):
            step(NSLOT_R * i + b, b)
        return 0

    lax.fori_loop(0, NG_R // NSLOT_R, outer, 0)
    for g in range(NG_R - PREF_R, NG_R):
        wait2(wr(g, g % NSLOT_R))


def _sc_gather_raw(msgA, msgB, a2b_flat):
    k = pl.kernel(
        _gather_raw_body,
        out_type=jax.ShapeDtypeStruct((ATOMS_PAD * MAX_NB, HIDDEN),
                                      jnp.float32),
        mesh=_sc_mesh(),
        scratch_types=(
            [pltpu.VMEM((HALF_W,), jnp.int32) for _ in range(2)]
            + [pltpu.VMEM((RCHUNK, HIDDEN), jnp.float32) for _ in range(8)]
            + [pltpu.SemaphoreType.DMA for _ in range(8)]
        ),
    )
    return k(msgA, msgB, a2b_flat)


_RBLK = 256


def _tc_reduce_body(nei_ref, out_ref):
    out_ref[...] = jnp.sum(nei_ref[...], axis=1)


def _tc_reduce(nei3d):
    return pl.pallas_call(
        _tc_reduce_body,
        grid=(ATOMS_PAD // _RBLK,),
        in_specs=[pl.BlockSpec((_RBLK, MAX_NB, HIDDEN), lambda i: (i, 0, 0))],
        out_specs=pl.BlockSpec((_RBLK, HIDDEN), lambda i: (i, 0)),
        out_shape=jax.ShapeDtypeStruct((ATOMS_PAD, HIDDEN), jnp.float32),
    )(nei3d)


def _sc_gather_sum(msgA, msgB, a2b_flat):
    nei = _sc_gather_raw(msgA, msgB, a2b_flat)
    return _tc_reduce(nei.reshape(ATOMS_PAD, MAX_NB, HIDDEN))


DCHUNK = 64
NSLOT_D = 4
PREF_D = 2
NG_D = 156
TAIL_OFF = BONDS_PER_W - DCHUNK


def _dual_gather_body(t1_hbm, idx1_hbm, t2_hbm, idx2_hbm, g1_hbm, g2_hbm,
                      i1_all, i2_all,
                      b1_0, b1_1, b1_2, b1_3, b2_0, b2_1, b2_2, b2_3,
                      gs0, gs1, gs2, gs3, ws0, ws1, ws2, ws3):
    wid = lax.axis_index("s") * 2 + lax.axis_index("c")
    bbase = wid * BONDS_PER_W
    pltpu.sync_copy(idx1_hbm.at[pl.ds(bbase, BONDS_PER_W)], i1_all)
    pltpu.sync_copy(idx2_hbm.at[pl.ds(bbase, BONDS_PER_W)], i2_all)

    b1s = (b1_0, b1_1, b1_2, b1_3)
    b2s = (b2_0, b2_1, b2_2, b2_3)
    gss = (gs0, gs1, gs2, gs3)
    wss = (ws0, ws1, ws2, ws3)

    def gath(off, b):
        return (pltpu.make_async_copy(t1_hbm.at[i1_all.at[pl.ds(off, DCHUNK)]],
                                      b1s[b], gss[b]),
                pltpu.make_async_copy(t2_hbm.at[i2_all.at[pl.ds(off, DCHUNK)]],
                                      b2s[b], gss[b]))

    def wr(off, b):
        o = bbase + off
        return (pltpu.make_async_copy(b1s[b], g1_hbm.at[pl.ds(o, DCHUNK)],
                                      wss[b]),
                pltpu.make_async_copy(b2s[b], g2_hbm.at[pl.ds(o, DCHUNK)],
                                      wss[b]))

    def start2(pair):
        pair[0].start()
        pair[1].start()

    def wait2(pair):
        pair[0].wait()
        pair[1].wait()

    for b in range(PREF_D):
        start2(gath(b * DCHUNK, b))

    def step(g, b):
        wait2(gath(g * DCHUNK, b))
        start2(wr(g * DCHUNK, b))
        f = g + PREF_D
        fb = (b + PREF_D) % NSLOT_D

        @pl.when(f >= NSLOT_D)
        def _():
            wait2(wr((f - NSLOT_D) * DCHUNK, fb))

        @pl.when(f < NG_D)
        def _():
            start2(gath(f * DCHUNK, fb))

    def outer(i, _):
        for b in range(NSLOT_D):
            step(NSLOT_D * i + b, b)
        return 0

    lax.fori_loop(0, NG_D // NSLOT_D, outer, 0)
    for g in range(NG_D - PREF_D, NG_D):
        wait2(wr(g * DCHUNK, g % NSLOT_D))
    start2(gath(TAIL_OFF, 0))
    wait2(gath(TAIL_OFF, 0))
    start2(wr(TAIL_OFF, 0))
    wait2(wr(TAIL_OFF, 0))


def _sc_dual_gather(t1, idx1, t2, idx2):
    k = pl.kernel(
        _dual_gather_body,
        out_type=(jax.ShapeDtypeStruct((N_BONDS, HIDDEN), jnp.float32),
                  jax.ShapeDtypeStruct((N_BONDS, HIDDEN), jnp.float32)),
        mesh=_sc_mesh(),
        scratch_types=(
            [pltpu.VMEM((BONDS_PER_W,), jnp.int32) for _ in range(2)]
            + [pltpu.VMEM((DCHUNK, HIDDEN), jnp.float32) for _ in range(8)]
            + [pltpu.SemaphoreType.DMA for _ in range(8)]
        ),
    )
    return k(t1, idx1, t2, idx2)


_BLK = 2000


def _tc1_body(fb_ref, wi_ref, wh_ref, inp_ref, m_ref, negmh_ref, negmh2_ref):
    inp = jnp.dot(fb_ref[...], wi_ref[...], preferred_element_type=jnp.float32)
    m = jnp.maximum(inp, 0.0)
    inp_ref[...] = inp
    m_ref[...] = m
    negmh = -jnp.dot(m, wh_ref[...], preferred_element_type=jnp.float32)
    negmh_ref[...] = negmh
    negmh2_ref[...] = negmh


def _tc1(f_bonds, W_i, W_h):
    return pl.pallas_call(
        _tc1_body,
        grid=(N_BONDS // _BLK,),
        in_specs=[
            pl.BlockSpec((_BLK, BOND_FDIM), lambda i: (i, 0)),
            pl.BlockSpec((BOND_FDIM, HIDDEN), lambda i: (0, 0)),
            pl.BlockSpec((HIDDEN, HIDDEN), lambda i: (0, 0)),
        ],
        out_specs=[
            pl.BlockSpec((_BLK, HIDDEN), lambda i: (i, 0)),
            pl.BlockSpec((_BLK, HIDDEN), lambda i: (i, 0)),
            pl.BlockSpec((_BLK, HIDDEN), lambda i: (i, 0)),
            pl.BlockSpec((_BLK, HIDDEN), lambda i: (i, 0)),
        ],
        out_shape=[
            jax.ShapeDtypeStruct((N_BONDS, HIDDEN), jnp.float32),
            jax.ShapeDtypeStruct((N_BONDS, HIDDEN), jnp.float32),
            jax.ShapeDtypeStruct((N_BONDS, HIDDEN), jnp.float32),
            jax.ShapeDtypeStruct((N_BONDS, HIDDEN), jnp.float32),
        ],
    )(f_bonds, W_i, W_h)


def _tc_small_body(am_ref, wh_ref, ah_ref):
    ah_ref[...] = jnp.dot(am_ref[...], wh_ref[...],
                          preferred_element_type=jnp.float32)


def _tc_small(amsg, W_h):
    return pl.pallas_call(
        _tc_small_body,
        grid=(ATOMS_PAD // 2048,),
        in_specs=[
            pl.BlockSpec((2048, HIDDEN), lambda i: (i, 0)),
            pl.BlockSpec((HIDDEN, HIDDEN), lambda i: (0, 0)),
        ],
        out_specs=pl.BlockSpec((2048, HIDDEN), lambda i: (i, 0)),
        out_shape=jax.ShapeDtypeStruct((ATOMS_PAD, HIDDEN), jnp.float32),
    )(amsg, W_h)


def _tc_iter_body(inp_ref, g1_ref, g2_ref, wh_ref, m_ref, negmh_ref,
                  negmh2_ref):
    m = jnp.maximum(inp_ref[...] - g1_ref[...] + g2_ref[...], 0.0)
    m_ref[...] = m
    negmh = -jnp.dot(m, wh_ref[...], preferred_element_type=jnp.float32)
    negmh_ref[...] = negmh
    negmh2_ref[...] = negmh


def _tc_iter(inp, g1, g2, W_h):
    return pl.pallas_call(
        _tc_iter_body,
        grid=(N_BONDS // _BLK,),
        in_specs=[
            pl.BlockSpec((_BLK, HIDDEN), lambda i: (i, 0)),
            pl.BlockSpec((_BLK, HIDDEN), lambda i: (i, 0)),
            pl.BlockSpec((_BLK, HIDDEN), lambda i: (i, 0)),
            pl.BlockSpec((HIDDEN, HIDDEN), lambda i: (0, 0)),
        ],
        out_specs=[
            pl.BlockSpec((_BLK, HIDDEN), lambda i: (i, 0)),
            pl.BlockSpec((_BLK, HIDDEN), lambda i: (i, 0)),
            pl.BlockSpec((_BLK, HIDDEN), lambda i: (i, 0)),
        ],
        out_shape=[
            jax.ShapeDtypeStruct((N_BONDS, HIDDEN), jnp.float32),
            jax.ShapeDtypeStruct((N_BONDS, HIDDEN), jnp.float32),
            jax.ShapeDtypeStruct((N_BONDS, HIDDEN), jnp.float32),
        ],
    )(inp, g1, g2, W_h)


def _tc_last_body(inp_ref, g1_ref, g2_ref, m_ref, m2_ref):
    m = jnp.maximum(inp_ref[...] - g1_ref[...] + g2_ref[...], 0.0)
    m_ref[...] = m
    m2_ref[...] = m


def _tc_last(inp, g1, g2):
    return pl.pallas_call(
        _tc_last_body,
        grid=(N_BONDS // _BLK,),
        in_specs=[
            pl.BlockSpec((_BLK, HIDDEN), lambda i: (i, 0)),
            pl.BlockSpec((_BLK, HIDDEN), lambda i: (i, 0)),
            pl.BlockSpec((_BLK, HIDDEN), lambda i: (i, 0)),
        ],
        out_specs=[
            pl.BlockSpec((_BLK, HIDDEN), lambda i: (i, 0)),
            pl.BlockSpec((_BLK, HIDDEN), lambda i: (i, 0)),
        ],
        out_shape=[
            jax.ShapeDtypeStruct((N_BONDS, HIDDEN), jnp.float32),
            jax.ShapeDtypeStruct((N_BONDS, HIDDEN), jnp.float32),
        ],
    )(inp, g1, g2)


_ABLK = 1000


def _tc_out_body(fa_ref, am_ref, seg_ref, wo1_ref, wo2_ref, bo_ref,
                 out_ref, sums_ref, cnts_ref):
    i = pl.program_id(0)

    @pl.when(i == 0)
    def _():
        sums_ref[...] = jnp.zeros_like(sums_ref)
        cnts_ref[...] = jnp.zeros_like(cnts_ref)

    hid = jnp.dot(fa_ref[...], wo1_ref[...], preferred_element_type=jnp.float32)
    hid = hid + jnp.dot(am_ref[...], wo2_ref[...],
                        preferred_element_type=jnp.float32)
    hid = jnp.maximum(hid + bo_ref[...], 0.0)

    seg = seg_ref[0]
    oh = (lax.broadcasted_iota(jnp.int32, (N_MOLS, _ABLK), 0) == seg
          ).astype(jnp.float32)
    sums_ref[...] += jnp.dot(oh, hid, preferred_element_type=jnp.float32)
    cnts_ref[...] += jnp.dot(oh, jnp.ones((_ABLK, HIDDEN), jnp.float32),
                             preferred_element_type=jnp.float32)

    @pl.when(i == pl.num_programs(0) - 1)
    def _():
        out_ref[...] = sums_ref[...] / jnp.maximum(cnts_ref[...], 1.0)


def _tc_out(f_atoms, amsg, seg3d, W_o1, W_o2, b_o2):
    return pl.pallas_call(
        _tc_out_body,
        grid=(N_ATOMS // _ABLK,),
        in_specs=[
            pl.BlockSpec((_ABLK, ATOM_FDIM), lambda i: (i, 0)),
            pl.BlockSpec((_ABLK, HIDDEN), lambda i: (i, 0)),
            pl.BlockSpec((1, 1, _ABLK), lambda i: (i, 0, 0)),
            pl.BlockSpec((ATOM_FDIM, HIDDEN), lambda i: (0, 0)),
            pl.BlockSpec((HIDDEN, HIDDEN), lambda i: (0, 0)),
            pl.BlockSpec((1, HIDDEN), lambda i: (0, 0)),
        ],
        out_specs=pl.BlockSpec((N_MOLS, HIDDEN), lambda i: (0, 0)),
        out_shape=jax.ShapeDtypeStruct((N_MOLS, HIDDEN), jnp.float32),
        scratch_shapes=[
            pltpu.VMEM((N_MOLS, HIDDEN), jnp.float32),
            pltpu.VMEM((N_MOLS, HIDDEN), jnp.float32),
        ],
    )(f_atoms, amsg, seg3d, W_o1, W_o2, b_o2)


@jax.jit
def kernel(f_atoms, f_bonds, a2b, b2a, b2revb, segment_ids, W_i, W_h, W_o, b_o):
    a2b_flat = jnp.pad(a2b, ((0, ATOMS_PAD - N_ATOMS), (0, 0))).reshape(-1)
    seg3d = segment_ids.reshape(10, 1, _ABLK)
    W_o1 = W_o[:ATOM_FDIM]
    W_o2 = W_o[ATOM_FDIM:]
    b_o2 = b_o.reshape(1, HIDDEN)

    inp, m, negMh, negMh2 = _tc1(f_bonds, W_i, W_h)
    for d in range(DEPTH - 1):
        negAh = _sc_gather_sum(negMh, negMh2, a2b_flat)
        g1, g2 = _sc_dual_gather(negAh, b2a, negMh, b2revb)
        if d < DEPTH - 2:
            m, negMh, negMh2 = _tc_iter(inp, g1, g2, W_h)
        else:
            m, m2 = _tc_last(inp, g1, g2)
    amsg = _sc_gather_sum(m, m2, a2b_flat)
    return _tc_out(f_atoms, amsg[:N_ATOMS], seg3d, W_o1, W_o2, b_o2)


# speedup vs baseline: 1.0872x; 1.0872x over previous
import functools

import jax
import jax.numpy as jnp
from jax import lax
from jax.experimental import pallas as pl
from jax.experimental.pallas import tpu as pltpu
from jax.experimental.pallas import tpu_sc as plsc

N_ATOMS = 10000
N_BONDS = 320000
MAX_NB = 32
ATOM_FDIM = 128
BOND_FDIM = 144
HIDDEN = 128
N_MOLS = 512
DEPTH = 3

NW = 32
ATOMS_PAD = 10240
ATOMS_PER_W = ATOMS_PAD // NW
GROUP_ATOMS = 2
GROUPS_SUM = ATOMS_PER_W // GROUP_ATOMS
GSUM_CHUNK = GROUP_ATOMS * MAX_NB
BONDS_PER_W = N_BONDS // NW
GCHUNK = 128
FULL_GROUPS = BONDS_PER_W // GCHUNK
LAST_OFF = BONDS_PER_W - GCHUNK


def _sc_mesh():
    return plsc.VectorSubcoreMesh(core_axis_name="c", subcore_axis_name="s")


IDX_PER_W = ATOMS_PER_W * MAX_NB


RCHUNK = 64
NSLOT_R = 4
PREF_R = 2
NG_R = IDX_PER_W // (2 * RCHUNK)


def _gather_raw_body(t1_hbm, t2_hbm, a2b_hbm, out_hbm, idx_all,
                     b1_0, b1_1, b1_2, b1_3, b2_0, b2_1, b2_2, b2_3,
                     gs0, gs1, gs2, gs3, ws0, ws1, ws2, ws3):
    wid = lax.axis_index("s") * 2 + lax.axis_index("c")
    rbase = wid * IDX_PER_W
    pltpu.sync_copy(a2b_hbm.at[pl.ds(rbase, IDX_PER_W)], idx_all)

    b1s = (b1_0, b1_1, b1_2, b1_3)
    b2s = (b2_0, b2_1, b2_2, b2_3)
    gss = (gs0, gs1, gs2, gs3)
    wss = (ws0, ws1, ws2, ws3)

    def gath(g, b):
        o = g * 2 * RCHUNK
        return (pltpu.make_async_copy(
                    t1_hbm.at[idx_all.at[pl.ds(o, RCHUNK)]], b1s[b], gss[b]),
                pltpu.make_async_copy(
                    t2_hbm.at[idx_all.at[pl.ds(o + RCHUNK, RCHUNK)]],
                    b2s[b], gss[b]))

    def wr(g, b):
        o = rbase + g * 2 * RCHUNK
        return (pltpu.make_async_copy(
                    b1s[b], out_hbm.at[pl.ds(o, RCHUNK)], wss[b]),
                pltpu.make_async_copy(
                    b2s[b], out_hbm.at[pl.ds(o + RCHUNK, RCHUNK)], wss[b]))

    def start2(pair):
        pair[0].start()
        pair[1].start()

    def wait2(pair):
        pair[0].wait()
        pair[1].wait()

    for b in range(PREF_R):
        start2(gath(b, b))

    def step(g, b):
        wait2(gath(g, b))
        start2(wr(g, b))
        f = g + PREF_R
        fb = (b + PREF_R) % NSLOT_R

        @pl.when(f >= NSLOT_R)
        def _():
            wait2(wr(f - NSLOT_R, fb))

        @pl.when(f < NG_R)
        def _():
            start2(gath(f, fb))

    def outer(i, _):
        for b in range(NSLOT_R):
            step(NSLOT_R * i + b, b)
        return 0

    lax.fori_loop(0, NG_R // NSLOT_R, outer, 0)
    for g in range(NG_R - PREF_R, NG_R):
        wait2(wr(g, g % NSLOT_R))


def _sc_gather_raw(msgA, msgB, a2b_flat):
    k = pl.kernel(
        _gather_raw_body,
        out_type=jax.ShapeDtypeStruct((ATOMS_PAD * MAX_NB, HIDDEN),
                                      jnp.float32),
        mesh=_sc_mesh(),
        scratch_types=(
            [pltpu.VMEM((IDX_PER_W,), jnp.int32)]
            + [pltpu.VMEM((RCHUNK, HIDDEN), jnp.float32) for _ in range(8)]
            + [pltpu.SemaphoreType.DMA for _ in range(8)]
        ),
    )
    return k(msgA, msgB, a2b_flat)


_RBLK = 256


def _tc_reduce_body(nei_ref, out_ref):
    out_ref[...] = jnp.sum(nei_ref[...], axis=1)


def _tc_reduce(nei3d):
    return pl.pallas_call(
        _tc_reduce_body,
        grid=(ATOMS_PAD // _RBLK,),
        in_specs=[pl.BlockSpec((_RBLK, MAX_NB, HIDDEN), lambda i: (i, 0, 0))],
        out_specs=pl.BlockSpec((_RBLK, HIDDEN), lambda i: (i, 0)),
        out_shape=jax.ShapeDtypeStruct((ATOMS_PAD, HIDDEN), jnp.float32),
    )(nei3d)


def _sc_gather_sum(msgA, msgB, a2b_flat):
    nei = _sc_gather_raw(msgA, msgB, a2b_flat)
    return _tc_reduce(nei.reshape(ATOMS_PAD, MAX_NB, HIDDEN))


DCHUNK = 64
NSLOT_D = 4
PREF_D = 2
NG_D = 156
TAIL_OFF = BONDS_PER_W - DCHUNK


def _dual_gather_body(t1_hbm, idx1_hbm, t2_hbm, idx2_hbm, g1_hbm, g2_hbm,
                      i1_all, i2_all,
                      b1_0, b1_1, b1_2, b1_3, b2_0, b2_1, b2_2, b2_3,
                      gs0, gs1, gs2, gs3, ws0, ws1, ws2, ws3):
    wid = lax.axis_index("s") * 2 + lax.axis_index("c")
    bbase = wid * BONDS_PER_W
    pltpu.sync_copy(idx1_hbm.at[pl.ds(bbase, BONDS_PER_W)], i1_all)
    pltpu.sync_copy(idx2_hbm.at[pl.ds(bbase, BONDS_PER_W)], i2_all)

    b1s = (b1_0, b1_1, b1_2, b1_3)
    b2s = (b2_0, b2_1, b2_2, b2_3)
    gss = (gs0, gs1, gs2, gs3)
    wss = (ws0, ws1, ws2, ws3)

    def gath(off, b):
        return (pltpu.make_async_copy(t1_hbm.at[i1_all.at[pl.ds(off, DCHUNK)]],
                                      b1s[b], gss[b]),
                pltpu.make_async_copy(t2_hbm.at[i2_all.at[pl.ds(off, DCHUNK)]],
                                      b2s[b], gss[b]))

    def wr(off, b):
        o = bbase + off
        return (pltpu.make_async_copy(b1s[b], g1_hbm.at[pl.ds(o, DCHUNK)],
                                      wss[b]),
                pltpu.make_async_copy(b2s[b], g2_hbm.at[pl.ds(o, DCHUNK)],
                                      wss[b]))

    def start2(pair):
        pair[0].start()
        pair[1].start()

    def wait2(pair):
        pair[0].wait()
        pair[1].wait()

    for b in range(PREF_D):
        start2(gath(b * DCHUNK, b))

    def step(g, b):
        wait2(gath(g * DCHUNK, b))
        start2(wr(g * DCHUNK, b))
        f = g + PREF_D
        fb = (b + PREF_D) % NSLOT_D

        @pl.when(f >= NSLOT_D)
        def _():
            wait2(wr((f - NSLOT_D) * DCHUNK, fb))

        @pl.when(f < NG_D)
        def _():
            start2(gath(f * DCHUNK, fb))

    def outer(i, _):
        for b in range(NSLOT_D):
            step(NSLOT_D * i + b, b)
        return 0

    lax.fori_loop(0, NG_D // NSLOT_D, outer, 0)
    for g in range(NG_D - PREF_D, NG_D):
        wait2(wr(g * DCHUNK, g % NSLOT_D))
    start2(gath(TAIL_OFF, 0))
    wait2(gath(TAIL_OFF, 0))
    start2(wr(TAIL_OFF, 0))
    wait2(wr(TAIL_OFF, 0))


def _sc_dual_gather(t1, idx1, t2, idx2):
    k = pl.kernel(
        _dual_gather_body,
        out_type=(jax.ShapeDtypeStruct((N_BONDS, HIDDEN), jnp.float32),
                  jax.ShapeDtypeStruct((N_BONDS, HIDDEN), jnp.float32)),
        mesh=_sc_mesh(),
        scratch_types=(
            [pltpu.VMEM((BONDS_PER_W,), jnp.int32) for _ in range(2)]
            + [pltpu.VMEM((DCHUNK, HIDDEN), jnp.float32) for _ in range(8)]
            + [pltpu.SemaphoreType.DMA for _ in range(8)]
        ),
    )
    return k(t1, idx1, t2, idx2)


_BLK = 2000


def _tc1_body(fb_ref, wi_ref, wh_ref, inp_ref, m_ref, negmh_ref, negmh2_ref):
    inp = jnp.dot(fb_ref[...], wi_ref[...], preferred_element_type=jnp.float32)
    m = jnp.maximum(inp, 0.0)
    inp_ref[...] = inp
    m_ref[...] = m
    negmh = -jnp.dot(m, wh_ref[...], preferred_element_type=jnp.float32)
    negmh_ref[...] = negmh
    negmh2_ref[...] = negmh


def _tc1(f_bonds, W_i, W_h):
    return pl.pallas_call(
        _tc1_body,
        grid=(N_BONDS // _BLK,),
        in_specs=[
            pl.BlockSpec((_BLK, BOND_FDIM), lambda i: (i, 0)),
            pl.BlockSpec((BOND_FDIM, HIDDEN), lambda i: (0, 0)),
            pl.BlockSpec((HIDDEN, HIDDEN), lambda i: (0, 0)),
        ],
        out_specs=[
            pl.BlockSpec((_BLK, HIDDEN), lambda i: (i, 0)),
            pl.BlockSpec((_BLK, HIDDEN), lambda i: (i, 0)),
            pl.BlockSpec((_BLK, HIDDEN), lambda i: (i, 0)),
            pl.BlockSpec((_BLK, HIDDEN), lambda i: (i, 0)),
        ],
        out_shape=[
            jax.ShapeDtypeStruct((N_BONDS, HIDDEN), jnp.float32),
            jax.ShapeDtypeStruct((N_BONDS, HIDDEN), jnp.float32),
            jax.ShapeDtypeStruct((N_BONDS, HIDDEN), jnp.float32),
            jax.ShapeDtypeStruct((N_BONDS, HIDDEN), jnp.float32),
        ],
    )(f_bonds, W_i, W_h)


def _tc_small_body(am_ref, wh_ref, ah_ref):
    ah_ref[...] = jnp.dot(am_ref[...], wh_ref[...],
                          preferred_element_type=jnp.float32)


def _tc_small(amsg, W_h):
    return pl.pallas_call(
        _tc_small_body,
        grid=(ATOMS_PAD // 2048,),
        in_specs=[
            pl.BlockSpec((2048, HIDDEN), lambda i: (i, 0)),
            pl.BlockSpec((HIDDEN, HIDDEN), lambda i: (0, 0)),
        ],
        out_specs=pl.BlockSpec((2048, HIDDEN), lambda i: (i, 0)),
        out_shape=jax.ShapeDtypeStruct((ATOMS_PAD, HIDDEN), jnp.float32),
    )(amsg, W_h)


def _tc_iter_body(inp_ref, g1_ref, g2_ref, wh_ref, negmh_ref, negmh2_ref):
    m = jnp.maximum(inp_ref[...] - g1_ref[...] + g2_ref[...], 0.0)
    negmh = -jnp.dot(m, wh_ref[...], preferred_element_type=jnp.float32)
    negmh_ref[...] = negmh
    negmh2_ref[...] = negmh


def _tc_iter(inp, g1, g2, W_h):
    return pl.pallas_call(
        _tc_iter_body,
        grid=(N_BONDS // _BLK,),
        in_specs=[
            pl.BlockSpec((_BLK, HIDDEN), lambda i: (i, 0)),
            pl.BlockSpec((_BLK, HIDDEN), lambda i: (i, 0)),
            pl.BlockSpec((_BLK, HIDDEN), lambda i: (i, 0)),
            pl.BlockSpec((HIDDEN, HIDDEN), lambda i: (0, 0)),
        ],
        out_specs=[
            pl.BlockSpec((_BLK, HIDDEN), lambda i: (i, 0)),
            pl.BlockSpec((_BLK, HIDDEN), lambda i: (i, 0)),
        ],
        out_shape=[
            jax.ShapeDtypeStruct((N_BONDS, HIDDEN), jnp.float32),
            jax.ShapeDtypeStruct((N_BONDS, HIDDEN), jnp.float32),
        ],
    )(inp, g1, g2, W_h)


def _tc_last_body(inp_ref, g1_ref, g2_ref, m_ref, m2_ref):
    m = jnp.maximum(inp_ref[...] - g1_ref[...] + g2_ref[...], 0.0)
    m_ref[...] = m
    m2_ref[...] = m


def _tc_last(inp, g1, g2):
    return pl.pallas_call(
        _tc_last_body,
        grid=(N_BONDS // _BLK,),
        in_specs=[
            pl.BlockSpec((_BLK, HIDDEN), lambda i: (i, 0)),
            pl.BlockSpec((_BLK, HIDDEN), lambda i: (i, 0)),
            pl.BlockSpec((_BLK, HIDDEN), lambda i: (i, 0)),
        ],
        out_specs=[
            pl.BlockSpec((_BLK, HIDDEN), lambda i: (i, 0)),
            pl.BlockSpec((_BLK, HIDDEN), lambda i: (i, 0)),
        ],
        out_shape=[
            jax.ShapeDtypeStruct((N_BONDS, HIDDEN), jnp.float32),
            jax.ShapeDtypeStruct((N_BONDS, HIDDEN), jnp.float32),
        ],
    )(inp, g1, g2)


_ABLK = 1000


def _tc_out_body(fa_ref, am_ref, seg_ref, wo1_ref, wo2_ref, bo_ref,
                 out_ref, sums_ref, cnts_ref):
    i = pl.program_id(0)

    @pl.when(i == 0)
    def _():
        sums_ref[...] = jnp.zeros_like(sums_ref)
        cnts_ref[...] = jnp.zeros_like(cnts_ref)

    hid = jnp.dot(fa_ref[...], wo1_ref[...], preferred_element_type=jnp.float32)
    hid = hid + jnp.dot(am_ref[...], wo2_ref[...],
                        preferred_element_type=jnp.float32)
    hid = jnp.maximum(hid + bo_ref[...], 0.0)

    seg = seg_ref[0]
    oh = (lax.broadcasted_iota(jnp.int32, (N_MOLS, _ABLK), 0) == seg
          ).astype(jnp.float32)
    sums_ref[...] += jnp.dot(oh, hid, preferred_element_type=jnp.float32)
    cnts_ref[...] += jnp.dot(oh, jnp.ones((_ABLK, HIDDEN), jnp.float32),
                             preferred_element_type=jnp.float32)

    @pl.when(i == pl.num_programs(0) - 1)
    def _():
        out_ref[...] = sums_ref[...] / jnp.maximum(cnts_ref[...], 1.0)


def _tc_out(f_atoms, amsg, seg3d, W_o1, W_o2, b_o2):
    return pl.pallas_call(
        _tc_out_body,
        grid=(N_ATOMS // _ABLK,),
        in_specs=[
            pl.BlockSpec((_ABLK, ATOM_FDIM), lambda i: (i, 0)),
            pl.BlockSpec((_ABLK, HIDDEN), lambda i: (i, 0)),
            pl.BlockSpec((1, 1, _ABLK), lambda i: (i, 0, 0)),
            pl.BlockSpec((ATOM_FDIM, HIDDEN), lambda i: (0, 0)),
            pl.BlockSpec((HIDDEN, HIDDEN), lambda i: (0, 0)),
            pl.BlockSpec((1, HIDDEN), lambda i: (0, 0)),
        ],
        out_specs=pl.BlockSpec((N_MOLS, HIDDEN), lambda i: (0, 0)),
        out_shape=jax.ShapeDtypeStruct((N_MOLS, HIDDEN), jnp.float32),
        scratch_shapes=[
            pltpu.VMEM((N_MOLS, HIDDEN), jnp.float32),
            pltpu.VMEM((N_MOLS, HIDDEN), jnp.float32),
        ],
    )(f_atoms, amsg, seg3d, W_o1, W_o2, b_o2)


@jax.jit
def kernel(f_atoms, f_bonds, a2b, b2a, b2revb, segment_ids, W_i, W_h, W_o, b_o):
    a2b_flat = jnp.pad(a2b, ((0, ATOMS_PAD - N_ATOMS), (0, 0))).reshape(-1)
    seg3d = segment_ids.reshape(10, 1, _ABLK)
    W_o1 = W_o[:ATOM_FDIM]
    W_o2 = W_o[ATOM_FDIM:]
    b_o2 = b_o.reshape(1, HIDDEN)

    inp, m, negMh, negMh2 = _tc1(f_bonds, W_i, W_h)
    for d in range(DEPTH - 1):
        negAh = _sc_gather_sum(negMh, negMh2, a2b_flat)
        g1, g2 = _sc_dual_gather(negAh, b2a, negMh, b2revb)
        if d < DEPTH - 2:
            negMh, negMh2 = _tc_iter(inp, g1, g2, W_h)
        else:
            m, m2 = _tc_last(inp, g1, g2)
    amsg = _sc_gather_sum(m, m2, a2b_flat)
    return _tc_out(f_atoms, amsg[:N_ATOMS], seg3d, W_o1, W_o2, b_o2)


# speedup vs baseline: 1.0941x; 1.0063x over previous
import functools

import jax
import jax.numpy as jnp
from jax import lax
from jax.experimental import pallas as pl
from jax.experimental.pallas import tpu as pltpu
from jax.experimental.pallas import tpu_sc as plsc

N_ATOMS = 10000
N_BONDS = 320000
MAX_NB = 32
ATOM_FDIM = 128
BOND_FDIM = 144
HIDDEN = 128
N_MOLS = 512
DEPTH = 3

NW = 32
ATOMS_PAD = 10240
ATOMS_PER_W = ATOMS_PAD // NW
GROUP_ATOMS = 2
GROUPS_SUM = ATOMS_PER_W // GROUP_ATOMS
GSUM_CHUNK = GROUP_ATOMS * MAX_NB
BONDS_PER_W = N_BONDS // NW
GCHUNK = 128
FULL_GROUPS = BONDS_PER_W // GCHUNK
LAST_OFF = BONDS_PER_W - GCHUNK


def _sc_mesh():
    return plsc.VectorSubcoreMesh(core_axis_name="c", subcore_axis_name="s")


IDX_PER_W = ATOMS_PER_W * MAX_NB


RCHUNK = 64
NSLOT_R = 5
PREF_R = 3
NG_R = IDX_PER_W // (2 * RCHUNK)


def _gather_raw_body(t1_hbm, t2_hbm, a2b_hbm, out_hbm, idx_all,
                     b1_0, b1_1, b1_2, b1_3, b1_4, b2_0, b2_1, b2_2, b2_3,
                     b2_4, gs0, gs1, gs2, gs3, gs4, ws0, ws1, ws2, ws3, ws4):
    wid = lax.axis_index("s") * 2 + lax.axis_index("c")
    rbase = wid * IDX_PER_W
    pltpu.sync_copy(a2b_hbm.at[pl.ds(rbase, IDX_PER_W)], idx_all)

    b1s = (b1_0, b1_1, b1_2, b1_3, b1_4)
    b2s = (b2_0, b2_1, b2_2, b2_3, b2_4)
    gss = (gs0, gs1, gs2, gs3, gs4)
    wss = (ws0, ws1, ws2, ws3, ws4)

    def gath(g, b):
        o = g * 2 * RCHUNK
        return (pltpu.make_async_copy(
                    t1_hbm.at[idx_all.at[pl.ds(o, RCHUNK)]], b1s[b], gss[b]),
                pltpu.make_async_copy(
                    t2_hbm.at[idx_all.at[pl.ds(o + RCHUNK, RCHUNK)]],
                    b2s[b], gss[b]))

    def wr(g, b):
        o = rbase + g * 2 * RCHUNK
        return (pltpu.make_async_copy(
                    b1s[b], out_hbm.at[pl.ds(o, RCHUNK)], wss[b]),
                pltpu.make_async_copy(
                    b2s[b], out_hbm.at[pl.ds(o + RCHUNK, RCHUNK)], wss[b]))

    def start2(pair):
        pair[0].start()
        pair[1].start()

    def wait2(pair):
        pair[0].wait()
        pair[1].wait()

    for b in range(PREF_R):
        start2(gath(b, b))

    def step(g, b):
        wait2(gath(g, b))
        start2(wr(g, b))
        f = g + PREF_R
        fb = (b + PREF_R) % NSLOT_R

        @pl.when(f >= NSLOT_R)
        def _():
            wait2(wr(f - NSLOT_R, fb))

        @pl.when(f < NG_R)
        def _():
            start2(gath(f, fb))

    def outer(i, _):
        for b in range(NSLOT_R):
            step(NSLOT_R * i + b, b)
        return 0

    lax.fori_loop(0, NG_R // NSLOT_R, outer, 0)
    for g in range(NG_R - (NSLOT_R - PREF_R), NG_R):
        wait2(wr(g, g % NSLOT_R))


def _sc_gather_raw(msgA, msgB, a2b_flat):
    k = pl.kernel(
        _gather_raw_body,
        out_type=jax.ShapeDtypeStruct((ATOMS_PAD * MAX_NB, HIDDEN),
                                      jnp.float32),
        mesh=_sc_mesh(),
        scratch_types=(
            [pltpu.VMEM((IDX_PER_W,), jnp.int32)]
            + [pltpu.VMEM((RCHUNK, HIDDEN), jnp.float32) for _ in range(10)]
            + [pltpu.SemaphoreType.DMA for _ in range(10)]
        ),
    )
    return k(msgA, msgB, a2b_flat)


_RBLK = 256


def _tc_reduce_body(nei_ref, out_ref):
    out_ref[...] = jnp.sum(nei_ref[...], axis=1)


def _tc_reduce(nei3d):
    return pl.pallas_call(
        _tc_reduce_body,
        grid=(ATOMS_PAD // _RBLK,),
        in_specs=[pl.BlockSpec((_RBLK, MAX_NB, HIDDEN), lambda i: (i, 0, 0))],
        out_specs=pl.BlockSpec((_RBLK, HIDDEN), lambda i: (i, 0)),
        out_shape=jax.ShapeDtypeStruct((ATOMS_PAD, HIDDEN), jnp.float32),
    )(nei3d)


def _sc_gather_sum(msgA, msgB, a2b_flat):
    nei = _sc_gather_raw(msgA, msgB, a2b_flat)
    return _tc_reduce(nei.reshape(ATOMS_PAD, MAX_NB, HIDDEN))


DCHUNK = 64
NSLOT_D = 4
PREF_D = 2
NG_D = 156
TAIL_OFF = BONDS_PER_W - DCHUNK


def _dual_gather_body(t1_hbm, idx1_hbm, t2_hbm, idx2_hbm, g1_hbm, g2_hbm,
                      i1_all, i2_all,
                      b1_0, b1_1, b1_2, b1_3, b2_0, b2_1, b2_2, b2_3,
                      gs0, gs1, gs2, gs3, ws0, ws1, ws2, ws3):
    wid = lax.axis_index("s") * 2 + lax.axis_index("c")
    bbase = wid * BONDS_PER_W
    pltpu.sync_copy(idx1_hbm.at[pl.ds(bbase, BONDS_PER_W)], i1_all)
    pltpu.sync_copy(idx2_hbm.at[pl.ds(bbase, BONDS_PER_W)], i2_all)

    b1s = (b1_0, b1_1, b1_2, b1_3)
    b2s = (b2_0, b2_1, b2_2, b2_3)
    gss = (gs0, gs1, gs2, gs3)
    wss = (ws0, ws1, ws2, ws3)

    def gath(off, b):
        return (pltpu.make_async_copy(t1_hbm.at[i1_all.at[pl.ds(off, DCHUNK)]],
                                      b1s[b], gss[b]),
                pltpu.make_async_copy(t2_hbm.at[i2_all.at[pl.ds(off, DCHUNK)]],
                                      b2s[b], gss[b]))

    def wr(off, b):
        o = bbase + off
        return (pltpu.make_async_copy(b1s[b], g1_hbm.at[pl.ds(o, DCHUNK)],
                                      wss[b]),
                pltpu.make_async_copy(b2s[b], g2_hbm.at[pl.ds(o, DCHUNK)],
                                      wss[b]))

    def start2(pair):
        pair[0].start()
        pair[1].start()

    def wait2(pair):
        pair[0].wait()
        pair[1].wait()

    for b in range(PREF_D):
        start2(gath(b * DCHUNK, b))

    def step(g, b):
        wait2(gath(g * DCHUNK, b))
        start2(wr(g * DCHUNK, b))
        f = g + PREF_D
        fb = (b + PREF_D) % NSLOT_D

        @pl.when(f >= NSLOT_D)
        def _():
            wait2(wr((f - NSLOT_D) * DCHUNK, fb))

        @pl.when(f < NG_D)
        def _():
            start2(gath(f * DCHUNK, fb))

    def outer(i, _):
        for b in range(NSLOT_D):
            step(NSLOT_D * i + b, b)
        return 0

    lax.fori_loop(0, NG_D // NSLOT_D, outer, 0)
    for g in range(NG_D - PREF_D, NG_D):
        wait2(wr(g * DCHUNK, g % NSLOT_D))
    start2(gath(TAIL_OFF, 0))
    wait2(gath(TAIL_OFF, 0))
    start2(wr(TAIL_OFF, 0))
    wait2(wr(TAIL_OFF, 0))


def _sc_dual_gather(t1, idx1, t2, idx2):
    k = pl.kernel(
        _dual_gather_body,
        out_type=(jax.ShapeDtypeStruct((N_BONDS, HIDDEN), jnp.float32),
                  jax.ShapeDtypeStruct((N_BONDS, HIDDEN), jnp.float32)),
        mesh=_sc_mesh(),
        scratch_types=(
            [pltpu.VMEM((BONDS_PER_W,), jnp.int32) for _ in range(2)]
            + [pltpu.VMEM((DCHUNK, HIDDEN), jnp.float32) for _ in range(8)]
            + [pltpu.SemaphoreType.DMA for _ in range(8)]
        ),
    )
    return k(t1, idx1, t2, idx2)


_BLK = 2000


def _tc1_body(fb_ref, wi_ref, wh_ref, inp_ref, m_ref, negmh_ref, negmh2_ref):
    inp = jnp.dot(fb_ref[...], wi_ref[...], preferred_element_type=jnp.float32)
    m = jnp.maximum(inp, 0.0)
    inp_ref[...] = inp
    m_ref[...] = m
    negmh = -jnp.dot(m, wh_ref[...], preferred_element_type=jnp.float32)
    negmh_ref[...] = negmh
    negmh2_ref[...] = negmh


def _tc1(f_bonds, W_i, W_h):
    return pl.pallas_call(
        _tc1_body,
        grid=(N_BONDS // _BLK,),
        in_specs=[
            pl.BlockSpec((_BLK, BOND_FDIM), lambda i: (i, 0)),
            pl.BlockSpec((BOND_FDIM, HIDDEN), lambda i: (0, 0)),
            pl.BlockSpec((HIDDEN, HIDDEN), lambda i: (0, 0)),
        ],
        out_specs=[
            pl.BlockSpec((_BLK, HIDDEN), lambda i: (i, 0)),
            pl.BlockSpec((_BLK, HIDDEN), lambda i: (i, 0)),
            pl.BlockSpec((_BLK, HIDDEN), lambda i: (i, 0)),
            pl.BlockSpec((_BLK, HIDDEN), lambda i: (i, 0)),
        ],
        out_shape=[
            jax.ShapeDtypeStruct((N_BONDS, HIDDEN), jnp.float32),
            jax.ShapeDtypeStruct((N_BONDS, HIDDEN), jnp.float32),
            jax.ShapeDtypeStruct((N_BONDS, HIDDEN), jnp.float32),
            jax.ShapeDtypeStruct((N_BONDS, HIDDEN), jnp.float32),
        ],
    )(f_bonds, W_i, W_h)


def _tc_small_body(am_ref, wh_ref, ah_ref):
    ah_ref[...] = jnp.dot(am_ref[...], wh_ref[...],
                          preferred_element_type=jnp.float32)


def _tc_small(amsg, W_h):
    return pl.pallas_call(
        _tc_small_body,
        grid=(ATOMS_PAD // 2048,),
        in_specs=[
            pl.BlockSpec((2048, HIDDEN), lambda i: (i, 0)),
            pl.BlockSpec((HIDDEN, HIDDEN), lambda i: (0, 0)),
        ],
        out_specs=pl.BlockSpec((2048, HIDDEN), lambda i: (i, 0)),
        out_shape=jax.ShapeDtypeStruct((ATOMS_PAD, HIDDEN), jnp.float32),
    )(amsg, W_h)


def _tc_iter_body(inp_ref, g1_ref, g2_ref, wh_ref, negmh_ref, negmh2_ref):
    m = jnp.maximum(inp_ref[...] - g1_ref[...] + g2_ref[...], 0.0)
    negmh = -jnp.dot(m, wh_ref[...], preferred_element_type=jnp.float32)
    negmh_ref[...] = negmh
    negmh2_ref[...] = negmh


def _tc_iter(inp, g1, g2, W_h):
    return pl.pallas_call(
        _tc_iter_body,
        grid=(N_BONDS // _BLK,),
        in_specs=[
            pl.BlockSpec((_BLK, HIDDEN), lambda i: (i, 0)),
            pl.BlockSpec((_BLK, HIDDEN), lambda i: (i, 0)),
            pl.BlockSpec((_BLK, HIDDEN), lambda i: (i, 0)),
            pl.BlockSpec((HIDDEN, HIDDEN), lambda i: (0, 0)),
        ],
        out_specs=[
            pl.BlockSpec((_BLK, HIDDEN), lambda i: (i, 0)),
            pl.BlockSpec((_BLK, HIDDEN), lambda i: (i, 0)),
        ],
        out_shape=[
            jax.ShapeDtypeStruct((N_BONDS, HIDDEN), jnp.float32),
            jax.ShapeDtypeStruct((N_BONDS, HIDDEN), jnp.float32),
        ],
    )(inp, g1, g2, W_h)


def _tc_last_body(inp_ref, g1_ref, g2_ref, m_ref, m2_ref):
    m = jnp.maximum(inp_ref[...] - g1_ref[...] + g2_ref[...], 0.0)
    m_ref[...] = m
    m2_ref[...] = m


def _tc_last(inp, g1, g2):
    return pl.pallas_call(
        _tc_last_body,
        grid=(N_BONDS // _BLK,),
        in_specs=[
            pl.BlockSpec((_BLK, HIDDEN), lambda i: (i, 0)),
            pl.BlockSpec((_BLK, HIDDEN), lambda i: (i, 0)),
            pl.BlockSpec((_BLK, HIDDEN), lambda i: (i, 0)),
        ],
        out_specs=[
            pl.BlockSpec((_BLK, HIDDEN), lambda i: (i, 0)),
            pl.BlockSpec((_BLK, HIDDEN), lambda i: (i, 0)),
        ],
        out_shape=[
            jax.ShapeDtypeStruct((N_BONDS, HIDDEN), jnp.float32),
            jax.ShapeDtypeStruct((N_BONDS, HIDDEN), jnp.float32),
        ],
    )(inp, g1, g2)


_ABLK = 1000


def _tc_out_body(fa_ref, am_ref, seg_ref, wo1_ref, wo2_ref, bo_ref,
                 out_ref, sums_ref, cnts_ref):
    i = pl.program_id(0)

    @pl.when(i == 0)
    def _():
        sums_ref[...] = jnp.zeros_like(sums_ref)
        cnts_ref[...] = jnp.zeros_like(cnts_ref)

    hid = jnp.dot(fa_ref[...], wo1_ref[...], preferred_element_type=jnp.float32)
    hid = hid + jnp.dot(am_ref[...], wo2_ref[...],
                        preferred_element_type=jnp.float32)
    hid = jnp.maximum(hid + bo_ref[...], 0.0)

    seg = seg_ref[0]
    oh = (lax.broadcasted_iota(jnp.int32, (N_MOLS, _ABLK), 0) == seg
          ).astype(jnp.float32)
    sums_ref[...] += jnp.dot(oh, hid, preferred_element_type=jnp.float32)
    cnts_ref[...] += jnp.dot(oh, jnp.ones((_ABLK, HIDDEN), jnp.float32),
                             preferred_element_type=jnp.float32)

    @pl.when(i == pl.num_programs(0) - 1)
    def _():
        out_ref[...] = sums_ref[...] / jnp.maximum(cnts_ref[...], 1.0)


def _tc_out(f_atoms, amsg, seg3d, W_o1, W_o2, b_o2):
    return pl.pallas_call(
        _tc_out_body,
        grid=(N_ATOMS // _ABLK,),
        in_specs=[
            pl.BlockSpec((_ABLK, ATOM_FDIM), lambda i: (i, 0)),
            pl.BlockSpec((_ABLK, HIDDEN), lambda i: (i, 0)),
            pl.BlockSpec((1, 1, _ABLK), lambda i: (i, 0, 0)),
            pl.BlockSpec((ATOM_FDIM, HIDDEN), lambda i: (0, 0)),
            pl.BlockSpec((HIDDEN, HIDDEN), lambda i: (0, 0)),
            pl.BlockSpec((1, HIDDEN), lambda i: (0, 0)),
        ],
        out_specs=pl.BlockSpec((N_MOLS, HIDDEN), lambda i: (0, 0)),
        out_shape=jax.ShapeDtypeStruct((N_MOLS, HIDDEN), jnp.float32),
        scratch_shapes=[
            pltpu.VMEM((N_MOLS, HIDDEN), jnp.float32),
            pltpu.VMEM((N_MOLS, HIDDEN), jnp.float32),
        ],
    )(f_atoms, amsg, seg3d, W_o1, W_o2, b_o2)


@jax.jit
def kernel(f_atoms, f_bonds, a2b, b2a, b2revb, segment_ids, W_i, W_h, W_o, b_o):
    a2b_flat = jnp.pad(a2b, ((0, ATOMS_PAD - N_ATOMS), (0, 0))).reshape(-1)
    seg3d = segment_ids.reshape(10, 1, _ABLK)
    W_o1 = W_o[:ATOM_FDIM]
    W_o2 = W_o[ATOM_FDIM:]
    b_o2 = b_o.reshape(1, HIDDEN)

    inp, m, negMh, negMh2 = _tc1(f_bonds, W_i, W_h)
    for d in range(DEPTH - 1):
        negAh = _sc_gather_sum(negMh, negMh2, a2b_flat)
        g1, g2 = _sc_dual_gather(negAh, b2a, negMh, b2revb)
        if d < DEPTH - 2:
            negMh, negMh2 = _tc_iter(inp, g1, g2, W_h)
        else:
            m, m2 = _tc_last(inp, g1, g2)
    amsg = _sc_gather_sum(m, m2, a2b_flat)
    return _tc_out(f_atoms, amsg[:N_ATOMS], seg3d, W_o1, W_o2, b_o2)
